# Initial kernel scaffold; baseline (speedup 1.0000x reference)
import jax
import jax.numpy as jnp
from jax.experimental import pallas as pl


def kernel(weights, x, expert_ids):
    raise NotImplementedError("write your pallas kernel here")


# speedup vs baseline: 4.6690x; 4.6690x over previous
import functools

import jax
import jax.numpy as jnp
from jax.experimental import pallas as pl
from jax.experimental.pallas import tpu as pltpu

E = 16
U = 1024
V = 1024
Z = 8192

TM = 128
MB = Z // TM
P = MB + E - 1


def _gemm_body(meta_ref, x_ref, w_ref, o_ref):
    p = pl.program_id(0)
    start = meta_ref[2, p]
    end = meta_ref[3, p]
    first = meta_ref[4, p]
    valid = meta_ref[5, p]

    @pl.when(valid == 1)
    def _():
        row = jax.lax.broadcasted_iota(jnp.int32, (TM, 1), 0)
        mask = ((row >= start) & (row < end)).astype(jnp.float32)
        xm = x_ref[...] * mask
        acc = jnp.dot(xm, w_ref[0], preferred_element_type=jnp.float32)

        @pl.when(first == 1)
        def _():
            o_ref[...] = acc

        @pl.when(first == 0)
        def _():
            o_ref[...] += acc


def _pair_metadata(ids):
    ids = ids.astype(jnp.int32)
    seg = jnp.searchsorted(ids, jnp.arange(E + 1, dtype=jnp.int32)).astype(jnp.int32)
    lo = ids[::TM]
    hi = ids[TM - 1 :: TM]
    span = hi - lo + 1
    offs = jnp.concatenate([jnp.zeros((1,), jnp.int32), jnp.cumsum(span)]).astype(jnp.int32)
    total = offs[MB]
    p = jnp.arange(P, dtype=jnp.int32)
    q = jnp.minimum(p, total - 1)
    m = (jnp.searchsorted(offs, q, side="right") - 1).astype(jnp.int32)
    e = lo[m] + q - offs[m]
    start = jnp.clip(seg[e] - m * TM, 0, TM)
    end = jnp.clip(seg[e + 1] - m * TM, 0, TM)
    valid = (p < total).astype(jnp.int32)
    firstf = ((p == offs[m]) & (p < total)).astype(jnp.int32)
    return jnp.stack([m, e, start, end, firstf, valid])


@jax.jit
def kernel(weights, x, expert_ids):
    meta = _pair_metadata(expert_ids)
    wr = weights.reshape(E, U, V)
    grid_spec = pltpu.PrefetchScalarGridSpec(
        num_scalar_prefetch=1,
        grid=(P,),
        in_specs=[
            pl.BlockSpec((TM, U), lambda p, meta: (meta[0, p], 0)),
            pl.BlockSpec((1, U, V), lambda p, meta: (meta[1, p], 0, 0)),
        ],
        out_specs=pl.BlockSpec((TM, V), lambda p, meta: (meta[0, p], 0)),
    )
    out = pl.pallas_call(
        _gemm_body,
        grid_spec=grid_spec,
        out_shape=jax.ShapeDtypeStruct((Z, V), jnp.float32),
        compiler_params=pltpu.CompilerParams(
            dimension_semantics=("arbitrary",),
        ),
    )(meta, x, wr)
    return out
